# baseline (speedup 1.0000x reference)
import functools

import jax
import jax.numpy as jnp
from jax.experimental import pallas as pl
from jax.experimental.pallas import tpu as pltpu

_NEG_INF = float("-inf")


def _moe_dense_kernel(x_ref, rw_ref, rb_ref, fc1w_ref, fc1b_ref,
                      fc2w_ref, fc2b_ref, out_ref, cw_ref, xb_ref,
                      *, n_experts):
    e = pl.program_id(0)

    @pl.when(e == 0)
    def _xcast():
        xb_ref[...] = x_ref[0].astype(jnp.bfloat16)

    @pl.when(e == 0)
    def _router():
        logits = jax.lax.dot_general(
            x_ref[0], rw_ref[...],
            dimension_numbers=(((1,), (1,)), ((), ())),
            preferred_element_type=jnp.float32,
        ) + rb_ref[...][None, :]
        col = jax.lax.broadcasted_iota(jnp.int32, logits.shape, 1)
        m1 = jnp.max(logits, axis=1, keepdims=True)
        i1 = jnp.min(jnp.where(logits == m1, col, n_experts), axis=1,
                     keepdims=True)
        l2 = jnp.where(col == i1, _NEG_INF, logits)
        m2 = jnp.max(l2, axis=1, keepdims=True)
        i2 = jnp.min(jnp.where(l2 == m2, col, n_experts), axis=1,
                     keepdims=True)
        b = jnp.exp(m2 - m1)
        w1 = 1.0 / (1.0 + b)
        w2 = 1.0 - w1
        cw_ref[...] = (w1 * (col == i1).astype(jnp.float32)
                       + w2 * (col == i2).astype(jnp.float32))

    col_t = jax.lax.broadcasted_iota(jnp.int32, cw_ref.shape, 1)
    cw = jnp.sum(jnp.where(col_t == e, cw_ref[...], 0.0), axis=1,
                 keepdims=True)

    h = (jax.lax.dot_general(
        xb_ref[...], fc1w_ref[0].astype(jnp.bfloat16),
        dimension_numbers=(((1,), (1,)), ((), ())),
        preferred_element_type=jnp.float32,
    ) + fc1b_ref[pl.ds(e, 1), :]).astype(jnp.bfloat16)
    hdim = h.shape[1] // 2
    h1 = h[:, :hdim]
    h2 = h[:, hdim:]
    g = h1 * jax.nn.sigmoid(h1) * h2
    y = jax.lax.dot_general(
        cw.astype(jnp.bfloat16) * g, fc2w_ref[0].astype(jnp.bfloat16),
        dimension_numbers=(((1,), (1,)), ((), ())),
        preferred_element_type=jnp.float32,
    ) + cw * fc2b_ref[pl.ds(e, 1), :]

    @pl.when(e == 0)
    def _init():
        out_ref[0] = y

    @pl.when(e != 0)
    def _acc():
        out_ref[0] += y


def kernel(x, router_w, router_b, fc1_w, fc1_b, fc2_w, fc2_b):
    B, T, D = x.shape
    E, H2, _ = fc1_w.shape

    return pl.pallas_call(
        functools.partial(_moe_dense_kernel, n_experts=E),
        grid=(E,),
        in_specs=[
            pl.BlockSpec((B, T, D), lambda e: (0, 0, 0)),
            pl.BlockSpec((E, D), lambda e: (0, 0)),
            pl.BlockSpec((E,), lambda e: (0,)),
            pl.BlockSpec((1, H2, D), lambda e: (e, 0, 0)),
            pl.BlockSpec((E, H2), lambda e: (0, 0)),
            pl.BlockSpec((1, D, H2 // 2), lambda e: (e, 0, 0)),
            pl.BlockSpec((E, D), lambda e: (0, 0)),
        ],
        out_specs=pl.BlockSpec((B, T, D), lambda e: (0, 0, 0)),
        out_shape=jax.ShapeDtypeStruct((B, T, D), x.dtype),
        scratch_shapes=[
            pltpu.VMEM((T, E), jnp.float32),
            pltpu.VMEM((T, D), jnp.bfloat16),
        ],
        compiler_params=pltpu.CompilerParams(
            dimension_semantics=("arbitrary",),
        ),
    )(x, router_w, router_b, fc1_w, fc1_b, fc2_w, fc2_b)


# speedup vs baseline: 1.0044x; 1.0044x over previous
import functools

import jax
import jax.numpy as jnp
from jax.experimental import pallas as pl
from jax.experimental.pallas import tpu as pltpu

_NEG_INF = float("-inf")


def _moe_dense_kernel(x_ref, rw_ref, rb_ref, fc1w_ref, fc1b_ref,
                      fc2w_ref, fc2b_ref, out_ref, cw_ref, *, n_experts):
    e = pl.program_id(0)

    @pl.when(e == 0)
    def _router():
        logits = jax.lax.dot_general(
            x_ref[0], rw_ref[...],
            dimension_numbers=(((1,), (1,)), ((), ())),
            preferred_element_type=jnp.float32,
        ) + rb_ref[...][None, :]
        col = jax.lax.broadcasted_iota(jnp.int32, logits.shape, 1)
        m1 = jnp.max(logits, axis=1, keepdims=True)
        i1 = jnp.min(jnp.where(logits == m1, col, n_experts), axis=1,
                     keepdims=True)
        l2 = jnp.where(col == i1, _NEG_INF, logits)
        m2 = jnp.max(l2, axis=1, keepdims=True)
        i2 = jnp.min(jnp.where(l2 == m2, col, n_experts), axis=1,
                     keepdims=True)
        b = jnp.exp(m2 - m1)
        w1 = 1.0 / (1.0 + b)
        w2 = 1.0 - w1
        cw_ref[...] = (w1 * (col == i1).astype(jnp.float32)
                       + w2 * (col == i2).astype(jnp.float32))

    col_t = jax.lax.broadcasted_iota(jnp.int32, cw_ref.shape, 1)
    cw = jnp.sum(jnp.where(col_t == e, cw_ref[...], 0.0), axis=1,
                 keepdims=True)

    h = (jax.lax.dot_general(
        x_ref[0].astype(jnp.bfloat16), fc1w_ref[0].astype(jnp.bfloat16),
        dimension_numbers=(((1,), (1,)), ((), ())),
        preferred_element_type=jnp.float32,
    ) + fc1b_ref[pl.ds(e, 1), :]).astype(jnp.bfloat16)
    hdim = h.shape[1] // 2
    h1 = h[:, :hdim]
    h2 = h[:, hdim:]
    g = h1 * jax.nn.sigmoid(h1) * h2
    y = jax.lax.dot_general(
        cw.astype(jnp.bfloat16) * g, fc2w_ref[0].astype(jnp.bfloat16),
        dimension_numbers=(((1,), (1,)), ((), ())),
        preferred_element_type=jnp.float32,
    ) + cw * fc2b_ref[pl.ds(e, 1), :]

    @pl.when(e == 0)
    def _init():
        out_ref[0] = y

    @pl.when(e != 0)
    def _acc():
        out_ref[0] += y


def kernel(x, router_w, router_b, fc1_w, fc1_b, fc2_w, fc2_b):
    B, T, D = x.shape
    E, H2, _ = fc1_w.shape

    return pl.pallas_call(
        functools.partial(_moe_dense_kernel, n_experts=E),
        grid=(E,),
        in_specs=[
            pl.BlockSpec((B, T, D), lambda e: (0, 0, 0)),
            pl.BlockSpec((E, D), lambda e: (0, 0)),
            pl.BlockSpec((E,), lambda e: (0,)),
            pl.BlockSpec((1, H2, D), lambda e: (e, 0, 0)),
            pl.BlockSpec((E, H2), lambda e: (0, 0)),
            pl.BlockSpec((1, D, H2 // 2), lambda e: (e, 0, 0)),
            pl.BlockSpec((E, D), lambda e: (0, 0)),
        ],
        out_specs=pl.BlockSpec((B, T, D), lambda e: (0, 0, 0)),
        out_shape=jax.ShapeDtypeStruct((B, T, D), x.dtype),
        scratch_shapes=[
            pltpu.VMEM((T, E), jnp.float32),
        ],
        compiler_params=pltpu.CompilerParams(
            dimension_semantics=("arbitrary",),
        ),
    )(x, router_w, router_b, fc1_w, fc1_b, fc2_w, fc2_b)
